# baseline (speedup 1.0000x reference)
import functools

import jax
import jax.numpy as jnp
from jax import lax
from jax.experimental import pallas as pl
from jax.experimental.pallas import tpu as pltpu
from jax.experimental.pallas import tpu_sc as plsc

_B = 1024
_H = 64
_NC = 2
_NS = 16
_NW = _NC * _NS
_BPW = _B // _NW
_LANES = 16

_BN = 2048

_sc_mesh = plsc.VectorSubcoreMesh(core_axis_name="c", subcore_axis_name="s")


@functools.partial(
    pl.kernel,
    mesh=_sc_mesh,
    out_type=jax.ShapeDtypeStruct((_B, 2 * _H), jnp.float32),
    scratch_types=[
        pltpu.VMEM((_BPW,), jnp.int32),
        pltpu.VMEM((_BPW,), jnp.int32),
        pltpu.VMEM((_BPW, 2 * _H), jnp.float32),
        pltpu.SemaphoreType.DMA,
    ],
)
def _sc_gather(table_hbm, idx_hbm, out_hbm, idx_v, pidx_v, rows_v, sem):
    wid = lax.axis_index("s") * _NC + lax.axis_index("c")
    base = wid * _BPW
    pltpu.sync_copy(idx_hbm.at[pl.ds(base, _BPW)], idx_v)

    @pl.loop(0, _BPW, step=_LANES)
    def _(i):
        slc = pl.ds(i, _LANES)
        pidx_v.at[slc][...] = lax.shift_right_logical(idx_v.at[slc][...], 1)

    pltpu.async_copy(table_hbm.at[pidx_v], rows_v, sem).wait()
    pltpu.sync_copy(rows_v, out_hbm.at[pl.ds(base, _BPW)])


def _mm_body(g_ref, xi_ref, w_ref, o_ref):
    odd = (xi_ref[...] & 1) == 1
    h = jnp.where(odd, g_ref[:, _H:], g_ref[:, :_H])
    o_ref[...] = lax.dot_general(
        h.astype(jnp.bfloat16),
        w_ref[...].astype(jnp.bfloat16),
        dimension_numbers=(((1,), (1,)), ((), ())),
        preferred_element_type=jnp.float32,
    )


def kernel(x, emb, W):
    xi = x.astype(jnp.int32)
    table = emb.reshape(emb.shape[0] // 2, 2 * _H)
    g = _sc_gather(table, xi)
    V = W.shape[0]
    logits = pl.pallas_call(
        _mm_body,
        grid=(pl.cdiv(V, _BN),),
        in_specs=[
            pl.BlockSpec((_B, 2 * _H), lambda i: (0, 0)),
            pl.BlockSpec((_B, 1), lambda i: (0, 0)),
            pl.BlockSpec((_BN, _H), lambda i: (i, 0)),
        ],
        out_specs=pl.BlockSpec((_B, _BN), lambda i: (0, i)),
        out_shape=jax.ShapeDtypeStruct((_B, V), jnp.float32),
        compiler_params=pltpu.CompilerParams(
            dimension_semantics=("parallel",),
        ),
    )(g, xi.reshape(_B, 1), W)
    return logits
